# baseline (speedup 1.0000x reference)
import functools

import jax
import jax.numpy as jnp
from jax.experimental import pallas as pl
from jax.experimental.pallas import tpu as pltpu

N = 4096
D_GATE = 2304
D_EXP = 2304
H = 1024
O = 512
E = 8
K = 2

BN = 1024


def _gate_body(gx_ref, gw_ref, gb_ref, w_ref):
    logits = jnp.dot(gx_ref[...], gw_ref[...],
                     preferred_element_type=jnp.float32) + gb_ref[...]
    iota = jax.lax.broadcasted_iota(jnp.int32, logits.shape, 1)
    m1 = jnp.max(logits, axis=-1, keepdims=True)
    i1 = jnp.min(jnp.where(logits == m1, iota, E), axis=-1, keepdims=True)
    l2 = jnp.where(iota == i1, -jnp.inf, logits)
    m2 = jnp.max(l2, axis=-1, keepdims=True)
    i2 = jnp.min(jnp.where(l2 == m2, iota, E), axis=-1, keepdims=True)
    e2 = jnp.exp(m2 - m1)
    denom = 1.0 + e2
    w1 = 1.0 / denom
    w2 = e2 / denom
    w_ref[...] = (jnp.where(iota == i1, w1, 0.0)
                  + jnp.where(iota == i2, w2, 0.0))


def _expert_body(wd_ref, x_ref, w1_ref, b1_ref, w2_ref, b2_ref, y_ref):
    e = pl.program_id(1)

    @pl.when(e == 0)
    def _():
        y_ref[...] = jnp.zeros_like(y_ref)

    h = jnp.maximum(
        jnp.dot(x_ref[...], w1_ref[0], preferred_element_type=jnp.float32)
        + b1_ref[0], 0.0)
    out = jnp.dot(h.astype(jnp.bfloat16), w2_ref[0],
                  preferred_element_type=jnp.float32) + b2_ref[0]
    wd = wd_ref[...]
    lane = jax.lax.broadcasted_iota(jnp.int32, wd.shape, 1)
    w_col = jnp.sum(jnp.where(lane == e, wd, 0.0), axis=1, keepdims=True)
    y_ref[...] += w_col * out


def kernel(x_expert_input, gate_input, gate_W, gate_b, W1, b1, W2, b2):
    w_dense = pl.pallas_call(
        _gate_body,
        grid=(N // BN,),
        in_specs=[
            pl.BlockSpec((BN, D_GATE), lambda i: (i, 0)),
            pl.BlockSpec((D_GATE, E), lambda i: (0, 0)),
            pl.BlockSpec((1, E), lambda i: (0, 0)),
        ],
        out_specs=pl.BlockSpec((BN, E), lambda i: (i, 0)),
        out_shape=jax.ShapeDtypeStruct((N, E), jnp.float32),
    )(gate_input, gate_W, gate_b.reshape(1, E))

    y = pl.pallas_call(
        _expert_body,
        grid=(N // BN, E),
        in_specs=[
            pl.BlockSpec((BN, E), lambda i, e: (i, 0)),
            pl.BlockSpec((BN, D_EXP), lambda i, e: (i, 0)),
            pl.BlockSpec((1, D_EXP, H), lambda i, e: (e, 0, 0)),
            pl.BlockSpec((1, 1, H), lambda i, e: (e, 0, 0)),
            pl.BlockSpec((1, H, O), lambda i, e: (e, 0, 0)),
            pl.BlockSpec((1, 1, O), lambda i, e: (e, 0, 0)),
        ],
        out_specs=pl.BlockSpec((BN, O), lambda i, e: (i, 0)),
        out_shape=jax.ShapeDtypeStruct((N, O), jnp.float32),
        compiler_params=pltpu.CompilerParams(
            dimension_semantics=("parallel", "arbitrary")),
    )(w_dense, x_expert_input.astype(jnp.bfloat16), W1.astype(jnp.bfloat16),
      b1.reshape(E, 1, H), W2.astype(jnp.bfloat16), b2.reshape(E, 1, O))
    return y


# speedup vs baseline: 1.1275x; 1.1275x over previous
import functools

import jax
import jax.numpy as jnp
from jax import lax
from jax.experimental import pallas as pl
from jax.experimental.pallas import tpu as pltpu
from jax.experimental.pallas import tpu_sc as plsc

N = 4096
D_GATE = 2304
D_EXP = 2304
H = 1024
O = 512
E = 8
K = 2

BN = 1024
BR = 256
P = N * K + E * BR
NB = P // BR

NC = 2
NS = 16
NW = NC * NS
TS = 32
TOK_W = N // NW
PAIR_W = (N * K) // NW
PS = 64


def _gate_body(gx_ref, gw_ref, gb_ref, idx_ref, w_ref):
    logits = jnp.dot(gx_ref[...], gw_ref[...],
                     preferred_element_type=jnp.float32) + gb_ref[...]
    iota = lax.broadcasted_iota(jnp.int32, logits.shape, 1)
    m1 = jnp.max(logits, axis=-1, keepdims=True)
    i1 = jnp.min(jnp.where(logits == m1, iota, E), axis=-1, keepdims=True)
    l2 = jnp.where(iota == i1, -jnp.inf, logits)
    m2 = jnp.max(l2, axis=-1, keepdims=True)
    i2 = jnp.min(jnp.where(l2 == m2, iota, E), axis=-1, keepdims=True)
    e2 = jnp.exp(m2 - m1)
    denom = 1.0 + e2
    idx_ref[...] = jnp.concatenate([i1, i2], axis=1)
    w_ref[...] = jnp.concatenate([1.0 / denom, e2 / denom], axis=1)


def _scatter_body(x_ref, d0_ref, d1_ref, out_ref, buf, i0, i1, sem0, sem1):
    wid = lax.axis_index("s") * NC + lax.axis_index("c")

    def chunk(c, carry):
        base = wid * TOK_W + c * TS
        pltpu.sync_copy(x_ref.at[pl.ds(base, TS)], buf)
        pltpu.sync_copy(d0_ref.at[pl.ds(base, TS)], i0)
        pltpu.sync_copy(d1_ref.at[pl.ds(base, TS)], i1)
        cp0 = pltpu.async_copy(buf, out_ref.at[i0], sem0)
        cp1 = pltpu.async_copy(buf, out_ref.at[i1], sem1)
        cp0.wait()
        cp1.wait()
        return carry

    lax.fori_loop(0, TOK_W // TS, chunk, 0)


def _mlp_body(be_ref, x_ref, w1_ref, b1_ref, w2_ref, b2_ref, o_ref):
    h = jnp.maximum(
        jnp.dot(x_ref[...], w1_ref[0], preferred_element_type=jnp.float32)
        + b1_ref[0], 0.0)
    o_ref[...] = jnp.dot(h, w2_ref[0],
                         preferred_element_type=jnp.float32) + b2_ref[0]


def _combine_gather_body(outg_ref, dst_ref, pairs_ref, buf, idxv, sem):
    wid = lax.axis_index("s") * NC + lax.axis_index("c")

    def chunk(c, carry):
        base = wid * PAIR_W + c * PS
        pltpu.sync_copy(dst_ref.at[pl.ds(base, PS)], idxv)
        pltpu.async_copy(outg_ref.at[idxv], buf, sem).wait()
        pltpu.sync_copy(buf, pairs_ref.at[pl.ds(base, PS)])
        return carry

    lax.fori_loop(0, PAIR_W // PS, chunk, 0)


def _final_body(p_ref, w_ref, y_ref):
    pv = p_ref[...]
    w = w_ref[...]
    y_ref[...] = w[:, 0:1] * pv[:, :O] + w[:, 1:2] * pv[:, O:]


def kernel(x_expert_input, gate_input, gate_W, gate_b, W1, b1, W2, b2):
    idx, wts = pl.pallas_call(
        _gate_body,
        grid=(N // BN,),
        in_specs=[
            pl.BlockSpec((BN, D_GATE), lambda i: (i, 0)),
            pl.BlockSpec((D_GATE, E), lambda i: (0, 0)),
            pl.BlockSpec((1, E), lambda i: (0, 0)),
        ],
        out_specs=[
            pl.BlockSpec((BN, K), lambda i: (i, 0)),
            pl.BlockSpec((BN, K), lambda i: (i, 0)),
        ],
        out_shape=[
            jax.ShapeDtypeStruct((N, K), jnp.int32),
            jax.ShapeDtypeStruct((N, K), jnp.float32),
        ],
    )(gate_input, gate_W, gate_b.reshape(1, E))

    flat_e = idx.reshape(-1)
    oh = (flat_e[:, None] == jnp.arange(E, dtype=jnp.int32)[None, :]).astype(
        jnp.int32)
    cum = jnp.cumsum(oh, axis=0)
    rank = jnp.sum(oh * cum, axis=1) - 1
    counts = cum[-1]
    used = (counts + BR - 1) // BR
    blk_end = jnp.cumsum(used)
    blk_off = blk_end - used
    dst = blk_off[flat_e] * BR + rank
    block_expert = jnp.clip(
        jnp.sum((jnp.arange(NB)[:, None] >= blk_end[None, :]).astype(jnp.int32),
                axis=1), 0, E - 1)
    dstp = dst.reshape(N, K)
    d0 = dstp[:, 0] + 0
    d1 = dstp[:, 1] + 0

    x_g = pl.kernel(
        _scatter_body,
        out_type=jax.ShapeDtypeStruct((P, D_EXP), jnp.float32),
        mesh=plsc.VectorSubcoreMesh(core_axis_name="c", subcore_axis_name="s"),
        scratch_types=[
            pltpu.VMEM((TS, D_EXP), jnp.float32),
            pltpu.VMEM((TS,), jnp.int32),
            pltpu.VMEM((TS,), jnp.int32),
            pltpu.SemaphoreType.DMA,
            pltpu.SemaphoreType.DMA,
        ],
    )(x_expert_input, d0, d1)

    out_g = pl.pallas_call(
        _mlp_body,
        grid_spec=pltpu.PrefetchScalarGridSpec(
            num_scalar_prefetch=1,
            grid=(NB,),
            in_specs=[
                pl.BlockSpec((BR, D_EXP), lambda i, be: (i, 0)),
                pl.BlockSpec((1, D_EXP, H), lambda i, be: (be[i], 0, 0)),
                pl.BlockSpec((1, 1, H), lambda i, be: (be[i], 0, 0)),
                pl.BlockSpec((1, H, O), lambda i, be: (be[i], 0, 0)),
                pl.BlockSpec((1, 1, O), lambda i, be: (be[i], 0, 0)),
            ],
            out_specs=pl.BlockSpec((BR, O), lambda i, be: (i, 0)),
        ),
        out_shape=jax.ShapeDtypeStruct((P, O), jnp.float32),
        compiler_params=pltpu.CompilerParams(
            dimension_semantics=("arbitrary",)),
    )(block_expert, x_g, W1, b1.reshape(E, 1, H), W2, b2.reshape(E, 1, O))

    pairs = pl.kernel(
        _combine_gather_body,
        out_type=jax.ShapeDtypeStruct((N * K, O), jnp.float32),
        mesh=plsc.VectorSubcoreMesh(core_axis_name="c", subcore_axis_name="s"),
        scratch_types=[
            pltpu.VMEM((PS, O), jnp.float32),
            pltpu.VMEM((PS,), jnp.int32),
            pltpu.SemaphoreType.DMA,
        ],
    )(out_g, dst)

    y = pl.pallas_call(
        _final_body,
        grid=(N // BN,),
        in_specs=[
            pl.BlockSpec((BN, K * O), lambda i: (i, 0)),
            pl.BlockSpec((BN, K), lambda i: (i, 0)),
        ],
        out_specs=pl.BlockSpec((BN, O), lambda i: (i, 0)),
        out_shape=jax.ShapeDtypeStruct((N, O), jnp.float32),
    )(pairs.reshape(N, K * O), wts)
    return y


# speedup vs baseline: 1.1928x; 1.0579x over previous
import functools

import jax
import jax.numpy as jnp
from jax import lax
from jax.experimental import pallas as pl
from jax.experimental.pallas import tpu as pltpu
from jax.experimental.pallas import tpu_sc as plsc

N = 4096
D_GATE = 2304
D_EXP = 2304
H = 1024
O = 512
E = 8
K = 2

BN = 1024
NBLK = N // BN
BR = 256
P = N * K + E * BR
NB = P // BR

NC = 2
NS = 16
NW = NC * NS
TS = 32
TOK_W = N // NW
PAIR_W = (N * K) // NW
PS = 64


def _gate_body(gx_ref, gw_ref, gb_ref, wts_ref, d01_ref, be_ref,
               run_ref, rank_sc, e_sc):
    i = pl.program_id(0)

    @pl.when(i == 0)
    def _():
        run_ref[...] = jnp.zeros_like(run_ref)

    logits = jnp.dot(gx_ref[...], gw_ref[...],
                     preferred_element_type=jnp.float32) + gb_ref[...]
    iota8 = lax.broadcasted_iota(jnp.int32, logits.shape, 1)
    m1 = jnp.max(logits, axis=-1, keepdims=True)
    i1 = jnp.min(jnp.where(logits == m1, iota8, E), axis=-1, keepdims=True)
    l2 = jnp.where(iota8 == i1, -jnp.inf, logits)
    m2 = jnp.max(l2, axis=-1, keepdims=True)
    i2 = jnp.min(jnp.where(l2 == m2, iota8, E), axis=-1, keepdims=True)
    e2 = jnp.exp(m2 - m1)
    denom = 1.0 + e2
    wts_ref[...] = jnp.concatenate([1.0 / denom, e2 / denom], axis=1)

    r_io = lax.broadcasted_iota(jnp.int32, (BN, BN), 0)
    c_io = lax.broadcasted_iota(jnp.int32, (BN, BN), 1)
    tri = (r_io >= c_io).astype(jnp.float32)

    run = run_ref[...]
    for kk, ik in ((0, i1), (1, i2)):
        oh = (iota8 == ik).astype(jnp.float32)
        cum = lax.dot(tri, oh, precision=lax.Precision.HIGHEST)
        cum_i = (cum + 0.5).astype(jnp.int32)
        rank = jnp.sum(jnp.where(iota8 == ik, cum_i - 1 + run, 0),
                       axis=1, keepdims=True)
        rank_sc[kk, pl.ds(i * BN, BN), :] = rank
        e_sc[kk, pl.ds(i * BN, BN), :] = ik
        run = run + lax.slice(cum_i, (BN - 1, 0), (BN, E))
    run_ref[...] = run

    @pl.when(i == NBLK - 1)
    def _():
        used = lax.shift_right_logical(run + (BR - 1), 8)
        r8 = lax.broadcasted_iota(jnp.int32, (E, E), 0)
        c8 = lax.broadcasted_iota(jnp.int32, (E, E), 1)
        inc = (r8 <= c8).astype(jnp.float32)
        blk_end = (lax.dot(used.astype(jnp.float32), inc,
                           precision=lax.Precision.HIGHEST)
                   + 0.5).astype(jnp.int32)
        base = (blk_end - used) * BR
        for kk in (0, 1):
            dstv = rank_sc[kk]
            e_arr = e_sc[kk]
            for e in range(E):
                base_e = lax.slice(base, (0, e), (1, e + 1))
                dstv = dstv + jnp.where(e_arr == e, base_e, 0)
            d01_ref[kk] = dstv
        bids = lax.broadcasted_iota(jnp.int32, (1, NB), 1)
        acc = jnp.zeros((1, NB), jnp.int32)
        for e in range(E):
            be_e = lax.slice(blk_end, (0, e), (1, e + 1))
            acc = acc + jnp.where(bids >= be_e, 1, 0)
        be_ref[...] = jnp.clip(acc, 0, E - 1)


def _scatter_body(x_ref, d01_ref, out_ref, buf, i0, i1, sem0, sem1):
    wid = lax.axis_index("s") * NC + lax.axis_index("c")

    def chunk(c, carry):
        base = wid * TOK_W + c * TS
        pltpu.sync_copy(x_ref.at[pl.ds(base, TS)], buf)
        pltpu.sync_copy(d01_ref.at[0, pl.ds(base, TS)], i0)
        pltpu.sync_copy(d01_ref.at[1, pl.ds(base, TS)], i1)
        cp0 = pltpu.async_copy(buf, out_ref.at[i0], sem0)
        cp1 = pltpu.async_copy(buf, out_ref.at[i1], sem1)
        cp0.wait()
        cp1.wait()
        return carry

    lax.fori_loop(0, TOK_W // TS, chunk, 0)


def _mlp_body(be_ref, x_ref, w1_ref, b1_ref, w2_ref, b2_ref, o_ref):
    h = jnp.maximum(
        jnp.dot(x_ref[...], w1_ref[0], preferred_element_type=jnp.float32)
        + b1_ref[0], 0.0)
    o_ref[...] = jnp.dot(h, w2_ref[0],
                         preferred_element_type=jnp.float32) + b2_ref[0]


def _combine_gather_body(outg_ref, dst_ref, pairs_ref, buf, idxv, sem):
    wid = lax.axis_index("s") * NC + lax.axis_index("c")

    def chunk(c, carry):
        base = wid * PAIR_W + c * PS
        pltpu.sync_copy(dst_ref.at[pl.ds(base, PS)], idxv)
        pltpu.async_copy(outg_ref.at[idxv], buf, sem).wait()
        pltpu.sync_copy(buf, pairs_ref.at[pl.ds(base, PS)])
        return carry

    lax.fori_loop(0, PAIR_W // PS, chunk, 0)


def _final_body(p0_ref, p1_ref, w_ref, y_ref):
    w = w_ref[...]
    y_ref[...] = w[:, 0:1] * p0_ref[...] + w[:, 1:2] * p1_ref[...]


def kernel(x_expert_input, gate_input, gate_W, gate_b, W1, b1, W2, b2):
    wts, d01, be = pl.pallas_call(
        _gate_body,
        grid=(NBLK,),
        in_specs=[
            pl.BlockSpec((BN, D_GATE), lambda i: (i, 0)),
            pl.BlockSpec((D_GATE, E), lambda i: (0, 0)),
            pl.BlockSpec((1, E), lambda i: (0, 0)),
        ],
        out_specs=[
            pl.BlockSpec((BN, K), lambda i: (i, 0)),
            pl.BlockSpec((K, N, 1), lambda i: (0, 0, 0)),
            pl.BlockSpec((1, NB), lambda i: (0, 0)),
        ],
        out_shape=[
            jax.ShapeDtypeStruct((N, K), jnp.float32),
            jax.ShapeDtypeStruct((K, N, 1), jnp.int32),
            jax.ShapeDtypeStruct((1, NB), jnp.int32),
        ],
        scratch_shapes=[
            pltpu.VMEM((1, E), jnp.int32),
            pltpu.VMEM((K, N, 1), jnp.int32),
            pltpu.VMEM((K, N, 1), jnp.int32),
        ],
    )(gate_input, gate_W, gate_b.reshape(1, E))
    d01_2d = d01.reshape(K, N)

    x_g = pl.kernel(
        _scatter_body,
        out_type=jax.ShapeDtypeStruct((P, D_EXP), jnp.float32),
        mesh=plsc.VectorSubcoreMesh(core_axis_name="c", subcore_axis_name="s"),
        scratch_types=[
            pltpu.VMEM((TS, D_EXP), jnp.float32),
            pltpu.VMEM((TS,), jnp.int32),
            pltpu.VMEM((TS,), jnp.int32),
            pltpu.SemaphoreType.DMA,
            pltpu.SemaphoreType.DMA,
        ],
    )(x_expert_input, d01_2d)

    out_g = pl.pallas_call(
        _mlp_body,
        grid_spec=pltpu.PrefetchScalarGridSpec(
            num_scalar_prefetch=1,
            grid=(NB,),
            in_specs=[
                pl.BlockSpec((BR, D_EXP), lambda i, be: (i, 0)),
                pl.BlockSpec((1, D_EXP, H), lambda i, be: (be[i], 0, 0)),
                pl.BlockSpec((1, 1, H), lambda i, be: (be[i], 0, 0)),
                pl.BlockSpec((1, H, O), lambda i, be: (be[i], 0, 0)),
                pl.BlockSpec((1, 1, O), lambda i, be: (be[i], 0, 0)),
            ],
            out_specs=pl.BlockSpec((BR, O), lambda i, be: (i, 0)),
        ),
        out_shape=jax.ShapeDtypeStruct((P, O), jnp.float32),
        compiler_params=pltpu.CompilerParams(
            dimension_semantics=("arbitrary",)),
    )(be.reshape(NB), x_g, W1, b1.reshape(E, 1, H), W2, b2.reshape(E, 1, O))

    pairs = pl.kernel(
        _combine_gather_body,
        out_type=jax.ShapeDtypeStruct((K * N, O), jnp.float32),
        mesh=plsc.VectorSubcoreMesh(core_axis_name="c", subcore_axis_name="s"),
        scratch_types=[
            pltpu.VMEM((PS, O), jnp.float32),
            pltpu.VMEM((PS,), jnp.int32),
            pltpu.SemaphoreType.DMA,
        ],
    )(out_g, d01_2d.reshape(K * N))

    y = pl.pallas_call(
        _final_body,
        grid=(N // BN,),
        in_specs=[
            pl.BlockSpec((BN, O), lambda i: (i, 0)),
            pl.BlockSpec((BN, O), lambda i: (i + N // BN, 0)),
            pl.BlockSpec((BN, K), lambda i: (i, 0)),
        ],
        out_specs=pl.BlockSpec((BN, O), lambda i: (i, 0)),
        out_shape=jax.ShapeDtypeStruct((N, O), jnp.float32),
    )(pairs, pairs, wts)
    return y
